# baseline (speedup 1.0000x reference)
import functools

import jax
import jax.numpy as jnp
from jax import lax
from jax.experimental import pallas as pl
from jax.experimental.pallas import tpu as pltpu
from jax.experimental.pallas import tpu_sc as plsc

N = 10000
D = 128
E = 320000
EPS = 1e-5

NC = 2
NS = 16
L = 16
NW = NC * NS
EPW = E // NW
CH = 128
NCH = 80
EPWP = NCH * CH
SEW = 2 * CH
RPT = 624
TAIL0 = RPT * NS
TAIL = N - TAIL0

_mesh = plsc.VectorSubcoreMesh(
    core_axis_name="c", subcore_axis_name="s", num_cores=NC, num_subcores=NS)


@functools.partial(
    pl.kernel,
    out_type=jax.ShapeDtypeStruct((NC, N, D), jnp.float32),
    mesh=_mesh,
    scratch_types=[
        pltpu.VMEM((4, CH), jnp.int32),
        pltpu.VMEM((4, CH), jnp.float32),
        pltpu.VMEM((NCH, CH), jnp.int32),
        pltpu.VMEM((CH, D), jnp.float32),
        pltpu.VMEM((CH, D), jnp.float32),
        pltpu.VMEM_SHARED((N, D), jnp.float32),
        pltpu.SemaphoreType.DMA,
        pltpu.SemaphoreType.DMA,
        pltpu.SemaphoreType.DMA,
        pltpu.SemaphoreType.DMA,
        pltpu.SemaphoreType.DMA,
        pltpu.SemaphoreType.DMA,
        pltpu.SemaphoreType.DMA,
        pltpu.SemaphoreType.DMA,
        pltpu.SemaphoreType.DMA,
    ],
)
def _sc_agg(h_hbm, src_hbm, ew_hbm, dst_hbm, zeros_hbm, out_hbm,
            srcb, ewb, dst_v, rows0, rows1, acc,
            sem_d, sb0, sb1, sb2, sb3, sg0, sg1, sc0, sc1):
    cid = lax.axis_index("c")
    sid = lax.axis_index("s")
    wid = sid * NC + cid
    sbs = (sb0, sb1, sb2, sb3)
    rows = (rows0, rows1)
    sgs = (sg0, sg1)
    scs = (sc0, sc1)

    def _seb_copies(c, s):
        base = pl.multiple_of((wid * NCH + c) * CH, 8)
        return (
            pltpu.make_async_copy(
                src_hbm.at[pl.ds(base, CH)], srcb.at[s], sbs[s]),
            pltpu.make_async_copy(
                ew_hbm.at[pl.ds(base, CH)], ewb.at[s], sbs[s]),
        )

    def _gather(c, p, s):
        del c
        return pltpu.make_async_copy(
            h_hbm.at[srcb.at[s]], rows[p], sgs[p])

    def _scatter(c, p):
        return pltpu.make_async_copy(rows[p], acc.at[dst_v.at[c]], scs[p])

    def _scale(rw, s):
        def rbody(r0i, carry):
            r0 = r0i * L
            wv = ewb[s, pl.ds(pl.multiple_of(r0, 8), L)]
            for i2 in range(L):
                w = jnp.full((L,), wv[i2], jnp.float32)
                for cc in range(D // L):
                    rw[r0 + i2, pl.ds(cc * L, L)] = (
                        rw[r0 + i2, pl.ds(cc * L, L)] * w)
            return carry

        lax.fori_loop(0, CH // L, rbody, 0)

    d_dst = pltpu.make_async_copy(dst_hbm.at[wid], dst_v, sem_d)
    d_dst.start()
    for c in range(4):
        for cp in _seb_copies(c, c):
            cp.start()

    pltpu.sync_copy(zeros_hbm.at[pl.ds(sid * RPT, RPT)],
                    acc.at[pl.ds(sid * RPT, RPT)])

    @pl.when(sid == 0)
    def _():
        pltpu.sync_copy(zeros_hbm.at[pl.ds(TAIL0, TAIL)],
                        acc.at[pl.ds(TAIL0, TAIL)])

    for cp in _seb_copies(0, 0):
        cp.wait()
    _gather(0, 0, 0).start()
    d_dst.wait()
    plsc.subcore_barrier()

    def body(i, carry):
        cbase = 4 * i
        for k in range(4):
            c = cbase + k
            p = k % 2
            s = k
            sn = (k + 1) % 4

            _gather(c, p, s).wait()
            _scale(rows[p], s)

            @pl.when(c + 4 < NCH)
            def _():
                for cp in _seb_copies(c + 4, s):
                    cp.start()

            @pl.when(c >= 1)
            def _():
                _scatter(c - 1, 1 - p).wait()

            @pl.when(c + 1 < NCH)
            def _():
                for cp in _seb_copies(c + 1, sn):
                    cp.wait()
                _gather(c + 1, 1 - p, sn).start()

            _scatter(c, p).start(add=True)
        return carry

    lax.fori_loop(0, NCH // 4, body, 0)

    _scatter(NCH - 1, 1).wait()
    plsc.subcore_barrier()
    pltpu.sync_copy(acc.at[pl.ds(sid * RPT, RPT)],
                    out_hbm.at[cid, pl.ds(sid * RPT, RPT)])

    @pl.when(sid == 0)
    def _():
        pltpu.sync_copy(acc.at[pl.ds(TAIL0, TAIL)],
                        out_hbm.at[cid, pl.ds(TAIL0, TAIL)])


def _tc_in_body(x_ref, w_ref, b_ref, o_ref):
    o_ref[...] = jnp.maximum(
        jnp.dot(x_ref[...], w_ref[...], preferred_element_type=jnp.float32)
        + b_ref[...], 0.0)


def _tc_mix_body(p_ref, h_ref, wrel_ref, brel_ref, wroot_ref, g_ref, be_ref,
                 o_ref, *, relu):
    agg = p_ref[0] + p_ref[1]
    t = (jnp.dot(agg, wrel_ref[...], preferred_element_type=jnp.float32)
         + brel_ref[...]
         + jnp.dot(h_ref[...], wroot_ref[...], preferred_element_type=jnp.float32))
    mean = jnp.mean(t, axis=0, keepdims=True)
    var = jnp.mean(jnp.square(t - mean), axis=0, keepdims=True)
    t = (t - mean) / jnp.sqrt(var + EPS) * g_ref[...] + be_ref[...]
    if relu:
        t = jnp.maximum(t, 0.0)
    o_ref[...] = t


_tc_in = pl.pallas_call(
    _tc_in_body, out_shape=jax.ShapeDtypeStruct((N, D), jnp.float32))


def _tc_mix(p, h, wrel, brel, wroot, gamma, beta, relu):
    body = functools.partial(_tc_mix_body, relu=relu)
    return pl.pallas_call(
        body, out_shape=jax.ShapeDtypeStruct((N, D), jnp.float32))(
            p, h, wrel, brel.reshape(1, D), wroot,
            gamma.reshape(1, D), beta.reshape(1, D))


def kernel(x, adj, features, W_in, b_in, W_rel1, b_rel1, W_root1,
           W_rel2, b_rel2, W_root2, gamma1, beta1):
    pad = ((0, 0), (0, EPWP - EPW))
    srcp = jnp.pad(adj[0].reshape(NW, EPW), pad).reshape(-1)
    ewp = jnp.pad(features.reshape(NW, EPW), pad).reshape(-1)
    dst = jnp.pad(adj[1].reshape(NW, EPW), pad).reshape(NW, NCH, CH)
    zeros = jnp.zeros((N, D), jnp.float32)

    h0 = _tc_in(x, W_in, b_in.reshape(1, D))
    p1 = _sc_agg(h0, srcp, ewp, dst, zeros)
    h1 = _tc_mix(p1, h0, W_rel1, b_rel1, W_root1, gamma1, beta1, relu=True)
    p2 = _sc_agg(h1, srcp, ewp, dst, zeros)
    out = _tc_mix(p2, h1, W_rel2, b_rel2, W_root2, gamma1, beta1, relu=False)
    return out
